# baseline (speedup 1.0000x reference)
import functools

import jax
import jax.numpy as jnp
from jax import lax
from jax.experimental import pallas as pl
from jax.experimental.pallas import tpu as pltpu
from jax.experimental.pallas import tpu_sc as plsc

B, N, D = 4096, 19, 512
R = 4
STARTS = (0, 7, 12, 17)
ENDS = (7, 12, 17, 19)
SCALES = (1.0 / 7.0, 1.0 / 5.0, 1.0 / 5.0, 1.0 / 2.0)

NC, NS = 2, 16
NW = NC * NS
BPW = B // NW
CB = 4
NCHUNK = BPW // CB
LANES = 16
DCHUNKS = D // LANES


def _compute_chunk(ibuf, obuf):
    for b in range(CB):
        def dbody(dc, _, b=b):
            off = dc * LANES
            v = [ibuf[b, c, pl.ds(off, LANES)] for c in range(N)]
            for r in range(R):
                acc = v[STARTS[r]]
                for c in range(STARTS[r] + 1, ENDS[r]):
                    acc = acc + v[c]
                obuf[b, r, pl.ds(off, LANES)] = acc * jnp.float32(SCALES[r])
            return _
        lax.fori_loop(0, DCHUNKS, dbody, None)


def _pool_body(x_hbm, out_hbm, in0, in1, ob, isem0, isem1):
    wid = lax.axis_index("s") * NC + lax.axis_index("c")
    base = wid * BPW

    def start_in(g, buf, sem):
        pltpu.async_copy(x_hbm.at[pl.ds(base + g * CB, CB)], buf, sem)

    def wait_in(g, buf, sem):
        pltpu.make_async_copy(x_hbm.at[pl.ds(base + g * CB, CB)], buf, sem).wait()

    start_in(0, in0, isem0)

    def gbody(h, _):
        g = h * 2
        wait_in(g, in0, isem0)

        @pl.when(g + 1 < NCHUNK)
        def _():
            start_in(g + 1, in1, isem1)

        _compute_chunk(in0, ob)
        pltpu.sync_copy(ob, out_hbm.at[pl.ds(base + g * CB, CB)])

        wait_in(g + 1, in1, isem1)

        @pl.when(g + 2 < NCHUNK)
        def _():
            start_in(g + 2, in0, isem0)

        _compute_chunk(in1, ob)
        pltpu.sync_copy(ob, out_hbm.at[pl.ds(base + (g + 1) * CB, CB)])
        return _

    lax.fori_loop(0, NCHUNK // 2, gbody, None)


_pool = functools.partial(
    pl.kernel,
    out_type=jax.ShapeDtypeStruct((B, R, D), jnp.float32),
    mesh=plsc.VectorSubcoreMesh(core_axis_name="c", subcore_axis_name="s"),
    scratch_types=[
        pltpu.VMEM((CB, N, D), jnp.float32),
        pltpu.VMEM((CB, N, D), jnp.float32),
        pltpu.VMEM((CB, R, D), jnp.float32),
        pltpu.SemaphoreType.DMA,
        pltpu.SemaphoreType.DMA,
    ],
)(_pool_body)


@jax.jit
def kernel(node_embeddings):
    return _pool(node_embeddings)
